# baseline (speedup 1.0000x reference)
import jax
import jax.numpy as jnp
from jax.experimental import pallas as pl
from jax.experimental.pallas import tpu as pltpu

_B, _T = 16, 4096
_HI_BITS = 1149239296


def _loss_kernel(x_ref, t16_ref, bnd_ref, lbnd_ref, lenc_ref,
                 lenb_ref, total_ref, spoof_ref, bdry_ref):
    x = x_ref[...]
    xr = pltpu.roll(x, 2 * _T - 1, 1)
    m = jnp.maximum(x, xr)
    lse = m + jnp.log(jnp.exp(x - m) + jnp.exp(xr - m))
    col8 = jax.lax.broadcasted_iota(jnp.int32, (_B, 2 * _T), 1)
    even = (col8 & 1) == 0
    cmask8 = ((col8 < 2 * lenc_ref[...]) & even).astype(jnp.float32)
    tsel = t16_ref[...] == 1
    sel_sum = (jnp.sum(x * cmask8)
               + jnp.sum(jnp.where(tsel, (xr - x) * cmask8, 0.0)))
    spoof_num = jnp.sum(lse * cmask8) - sel_sum
    spoof = spoof_num / (jnp.sum(cmask8) + 1e-8)

    col = jax.lax.broadcasted_iota(jnp.int32, (_B, _T), 1)
    pred = bnd_ref[...]
    tgt = lbnd_ref[...].astype(jnp.float32)
    bmask = (col < lenb_ref[...]).astype(jnp.float32)
    lp = jnp.maximum(jnp.log(pred), -100.0)
    l1m = jnp.maximum(jnp.log(1.0 - pred), -100.0)
    loss = -(tgt * lp + (1.0 - tgt) * l1m) * bmask
    tgt_m = tgt * bmask
    pos = (tgt_m == 1.0).astype(jnp.float32)
    pos_count = jnp.sum(pos)
    neg_count_all = jnp.float32(_B * _T) - pos_count
    k = jnp.minimum(neg_count_all, jnp.floor(pos_count * 5.0))
    pos_loss = jnp.sum(loss * pos)
    neg_vals = loss * (1.0 - pos)

    vbits = jax.lax.bitcast_convert_type(neg_vals, jnp.int32)
    k_i = k.astype(jnp.int32)

    def body(_, carry):
        lo, hi = carry
        mid = lo + (hi - lo + 1) // 2
        cnt = jnp.sum((vbits >= mid).astype(jnp.int32))
        take = cnt >= k_i
        return jnp.where(take, mid, lo), jnp.where(take, hi, mid - 1)

    lo, _ = jax.lax.fori_loop(
        0, 31, body, (jnp.int32(0), jnp.int32(_HI_BITS)))

    t = jax.lax.bitcast_convert_type(lo, jnp.float32)
    gt = vbits > lo
    cnt_gt = jnp.sum(gt.astype(jnp.float32))
    sum_gt = jnp.sum(jnp.where(gt, neg_vals, 0.0))
    neg_loss = sum_gt + (k - cnt_gt) * t

    balanced = (pos_loss + neg_loss) / (pos_count + k + 1e-8)
    mean_loss = jnp.sum(loss) / jnp.float32(_B * _T)
    bdry = jnp.where(pos_count == 0.0, mean_loss, balanced)

    total_ref[...] = jnp.broadcast_to(spoof + 0.5 * bdry, (1, 1))
    spoof_ref[...] = jnp.broadcast_to(spoof, (1, 1))
    bdry_ref[...] = jnp.broadcast_to(bdry, (1, 1))


@jax.jit
def kernel(output, boundary, label_cls, label_boundary, len_cls, len_boundary):
    x = output.reshape(_B, 2 * _T)
    t16 = jax.lax.bitcast_convert_type(
        label_cls, jnp.int16).reshape(_B, 2 * _T)
    lenc = len_cls.reshape(_B, 1)
    lenb = len_boundary.reshape(_B, 1)
    total, spoof, bdry = pl.pallas_call(
        _loss_kernel,
        out_shape=[jax.ShapeDtypeStruct((1, 1), jnp.float32)] * 3,
    )(x, t16, boundary, label_boundary, lenc, lenb)
    return (total.reshape(()), spoof.reshape(()), bdry.reshape(()))


# speedup vs baseline: 1.8757x; 1.8757x over previous
import jax
import jax.numpy as jnp
from jax.experimental import pallas as pl

_B, _T = 16, 4096
_HI_BITS = 1149239296


def _loss_kernel(a_ref, b_ref, lcls_ref, bnd_ref, lbnd_ref, lenc_ref,
                 lenb_ref, total_ref, spoof_ref, bdry_ref):
    col = jax.lax.broadcasted_iota(jnp.int32, (_B, _T), 1)

    a = a_ref[...]
    b = b_ref[...]
    m = jnp.maximum(a, b)
    lse = m + jnp.log(jnp.exp(a - m) + jnp.exp(b - m))
    sel = jnp.where(lcls_ref[...] == 0, a, b)
    ce = lse - sel
    cmask = (col < lenc_ref[...]).astype(jnp.float32)
    spoof = jnp.sum(ce * cmask) / (jnp.sum(cmask) + 1e-8)

    pred = bnd_ref[...]
    tgt = lbnd_ref[...].astype(jnp.float32)
    bmask = (col < lenb_ref[...]).astype(jnp.float32)
    lp = jnp.maximum(jnp.log(pred), -100.0)
    l1m = jnp.maximum(jnp.log(1.0 - pred), -100.0)
    loss = -(tgt * lp + (1.0 - tgt) * l1m) * bmask
    tgt_m = tgt * bmask
    pos = (tgt_m == 1.0).astype(jnp.float32)
    pos_count = jnp.sum(pos)
    neg_count_all = jnp.float32(_B * _T) - pos_count
    k = jnp.minimum(neg_count_all, jnp.floor(pos_count * 5.0))
    pos_loss = jnp.sum(loss * pos)
    neg_vals = loss * (1.0 - pos)

    vbits = jax.lax.bitcast_convert_type(neg_vals, jnp.int32)
    k_i = k.astype(jnp.int32)

    def body(_, carry):
        lo, hi = carry
        mid = lo + (hi - lo + 1) // 2
        cnt = jnp.sum((vbits >= mid).astype(jnp.int32))
        take = cnt >= k_i
        return jnp.where(take, mid, lo), jnp.where(take, hi, mid - 1)

    lo, _ = jax.lax.fori_loop(
        0, 31, body, (jnp.int32(0), jnp.int32(_HI_BITS)))

    t = jax.lax.bitcast_convert_type(lo, jnp.float32)
    gt = vbits > lo
    cnt_gt = jnp.sum(gt.astype(jnp.float32))
    sum_gt = jnp.sum(jnp.where(gt, neg_vals, 0.0))
    neg_loss = sum_gt + (k - cnt_gt) * t

    balanced = (pos_loss + neg_loss) / (pos_count + k + 1e-8)
    mean_loss = jnp.sum(loss) / jnp.float32(_B * _T)
    bdry = jnp.where(pos_count == 0.0, mean_loss, balanced)

    total_ref[...] = jnp.broadcast_to(spoof + 0.5 * bdry, (1, 1))
    spoof_ref[...] = jnp.broadcast_to(spoof, (1, 1))
    bdry_ref[...] = jnp.broadcast_to(bdry, (1, 1))


@jax.jit
def kernel(output, boundary, label_cls, label_boundary, len_cls, len_boundary):
    a = output[:, :, 0]
    b = output[:, :, 1]
    lenc = len_cls.reshape(_B, 1)
    lenb = len_boundary.reshape(_B, 1)
    total, spoof, bdry = pl.pallas_call(
        _loss_kernel,
        out_shape=[jax.ShapeDtypeStruct((1, 1), jnp.float32)] * 3,
    )(a, b, label_cls, boundary, label_boundary, lenc, lenb)
    return (total.reshape(()), spoof.reshape(()), bdry.reshape(()))


# speedup vs baseline: 2.0308x; 1.0827x over previous
import jax
import jax.numpy as jnp
from jax.experimental import pallas as pl

_B, _T = 16, 4096
_HI_BITS = 1149239296


def _loss_kernel(a_ref, b_ref, lcls_ref, bnd_ref, lbnd_ref, lenc_ref,
                 lenb_ref, total_ref, spoof_ref, bdry_ref):
    col = jax.lax.broadcasted_iota(jnp.int32, (_B, _T), 1)

    a = a_ref[...]
    b = b_ref[...]
    m = jnp.maximum(a, b)
    lse = m + jnp.log(jnp.exp(a - m) + jnp.exp(b - m))
    sel = jnp.where(lcls_ref[...] == 0, a, b)
    ce = lse - sel
    cmask = (col < lenc_ref[...]).astype(jnp.float32)
    spoof = jnp.sum(ce * cmask) / (jnp.sum(cmask) + 1e-8)

    pred = bnd_ref[...]
    tgt = lbnd_ref[...].astype(jnp.float32)
    bmask = (col < lenb_ref[...]).astype(jnp.float32)
    selp = jnp.where(tgt == 1.0, pred, 1.0 - pred)
    loss = jnp.minimum(-jnp.log(selp), 100.0) * bmask
    tgt_m = tgt * bmask
    pos = (tgt_m == 1.0).astype(jnp.float32)
    pos_count = jnp.sum(pos)
    neg_count_all = jnp.float32(_B * _T) - pos_count
    k = jnp.minimum(neg_count_all, jnp.floor(pos_count * 5.0))
    pos_loss = jnp.sum(loss * pos)
    neg_vals = loss * (1.0 - pos)

    vbits = jax.lax.bitcast_convert_type(neg_vals, jnp.int32)
    k_i = k.astype(jnp.int32)

    def radix_round(lo, s, njs):
        t = jnp.int32(0)
        for j in range(1, njs + 1):
            m = lo + (j << s)
            c = jnp.sum((vbits >= m).astype(jnp.int32))
            t = t + (c >= k_i).astype(jnp.int32)
        return lo + t * (1 << s)

    lo = jnp.int32(0)
    for s in (27, 23, 19, 15, 11, 7, 3):
        lo = radix_round(lo, s, 15)
    lo = radix_round(lo, 0, 7)

    t = jax.lax.bitcast_convert_type(lo, jnp.float32)
    gt = vbits > lo
    cnt_gt = jnp.sum(gt.astype(jnp.float32))
    sum_gt = jnp.sum(jnp.where(gt, neg_vals, 0.0))
    neg_loss = jnp.where(k_i == 0, 0.0, sum_gt + (k - cnt_gt) * t)

    balanced = (pos_loss + neg_loss) / (pos_count + k + 1e-8)
    mean_loss = jnp.sum(loss) / jnp.float32(_B * _T)
    bdry = jnp.where(pos_count == 0.0, mean_loss, balanced)

    total_ref[...] = jnp.broadcast_to(spoof + 0.5 * bdry, (1, 1))
    spoof_ref[...] = jnp.broadcast_to(spoof, (1, 1))
    bdry_ref[...] = jnp.broadcast_to(bdry, (1, 1))


@jax.jit
def kernel(output, boundary, label_cls, label_boundary, len_cls, len_boundary):
    a = output[:, :, 0]
    b = output[:, :, 1]
    lenc = len_cls.reshape(_B, 1)
    lenb = len_boundary.reshape(_B, 1)
    total, spoof, bdry = pl.pallas_call(
        _loss_kernel,
        out_shape=[jax.ShapeDtypeStruct((1, 1), jnp.float32)] * 3,
    )(a, b, label_cls, boundary, label_boundary, lenc, lenb)
    return (total.reshape(()), spoof.reshape(()), bdry.reshape(()))


# speedup vs baseline: 2.1511x; 1.0592x over previous
import jax
import jax.numpy as jnp
from jax.experimental import pallas as pl

_B, _T = 16, 4096
_HI_BITS = 1149239296


def _loss_kernel(a_ref, b_ref, lcls_ref, bnd_ref, lbnd_ref, lenc_ref,
                 lenb_ref, total_ref, spoof_ref, bdry_ref):
    col = jax.lax.broadcasted_iota(jnp.int32, (_B, _T), 1)

    a = a_ref[...]
    b = b_ref[...]
    m = jnp.maximum(a, b)
    lse = m + jnp.log(jnp.exp(a - m) + jnp.exp(b - m))
    sel = jnp.where(lcls_ref[...] == 0, a, b)
    ce = lse - sel
    cmask = (col < lenc_ref[...]).astype(jnp.float32)
    spoof = jnp.sum(ce * cmask) / (jnp.sum(cmask) + 1e-8)

    pred = bnd_ref[...]
    tgt = lbnd_ref[...].astype(jnp.float32)
    bmask = (col < lenb_ref[...]).astype(jnp.float32)
    selp = jnp.where(tgt == 1.0, pred, 1.0 - pred)
    loss = jnp.minimum(-jnp.log(selp), 100.0) * bmask
    tgt_m = tgt * bmask
    pos = (tgt_m == 1.0).astype(jnp.float32)
    pos_count = jnp.sum(pos)
    neg_count_all = jnp.float32(_B * _T) - pos_count
    k = jnp.minimum(neg_count_all, jnp.floor(pos_count * 5.0))
    pos_loss = jnp.sum(loss * pos)
    neg_vals = loss * (1.0 - pos)

    vbits = jax.lax.bitcast_convert_type(neg_vals, jnp.int32)
    k_i = k.astype(jnp.int32)

    def radix_round(lo, s, njs):
        t = jnp.int32(0)
        for j in range(1, njs + 1):
            m = lo + (j << s)
            c = jnp.sum((vbits >= m).astype(jnp.int32))
            t = t + (c >= k_i).astype(jnp.int32)
        return lo + t * (1 << s)

    lo = jnp.int32(0)
    for s in (28, 25, 22, 19, 16, 13, 10, 7, 4, 1):
        lo = radix_round(lo, s, 7)
    lo = radix_round(lo, 0, 1)

    t = jax.lax.bitcast_convert_type(lo, jnp.float32)
    gt = vbits > lo
    cnt_gt = jnp.sum(gt.astype(jnp.float32))
    sum_gt = jnp.sum(jnp.where(gt, neg_vals, 0.0))
    neg_loss = jnp.where(k_i == 0, 0.0, sum_gt + (k - cnt_gt) * t)

    balanced = (pos_loss + neg_loss) / (pos_count + k + 1e-8)
    mean_loss = jnp.sum(loss) / jnp.float32(_B * _T)
    bdry = jnp.where(pos_count == 0.0, mean_loss, balanced)

    total_ref[...] = jnp.broadcast_to(spoof + 0.5 * bdry, (1, 1))
    spoof_ref[...] = jnp.broadcast_to(spoof, (1, 1))
    bdry_ref[...] = jnp.broadcast_to(bdry, (1, 1))


@jax.jit
def kernel(output, boundary, label_cls, label_boundary, len_cls, len_boundary):
    a = output[:, :, 0]
    b = output[:, :, 1]
    lenc = len_cls.reshape(_B, 1)
    lenb = len_boundary.reshape(_B, 1)
    total, spoof, bdry = pl.pallas_call(
        _loss_kernel,
        out_shape=[jax.ShapeDtypeStruct((1, 1), jnp.float32)] * 3,
    )(a, b, label_cls, boundary, label_boundary, lenc, lenb)
    return (total.reshape(()), spoof.reshape(()), bdry.reshape(()))
